# Initial kernel scaffold; baseline (speedup 1.0000x reference)
import jax
import jax.numpy as jnp
from jax.experimental import pallas as pl


def kernel(hidden_states, router_w, gate_w, up_w, down_w, sh_gate_w, sh_up_w, sh_down_w, shared_gate_w):
    raise NotImplementedError("write your pallas kernel here")


# speedup vs baseline: 1.0321x; 1.0321x over previous
import functools

import jax
import jax.numpy as jnp
from jax import lax
from jax.experimental import pallas as pl
from jax.experimental.pallas import tpu as pltpu

E = 8
TOPK = 2
D = 768
DFF = 2048
T = 2048

NE = E + 1
LANES = 128
DFF_BLK = 512
K_CHUNKS = DFF // DFF_BLK


def _router_kernel(x_ref, rw_ref, w_ref):
    x = x_ref[...]
    logits = lax.dot_general(
        x, rw_ref[...], (((1,), (1,)), ((), ())),
        preferred_element_type=jnp.float32)
    cols = lax.broadcasted_iota(jnp.int32, (T, LANES), 1)
    valid = cols < E
    masked = jnp.where(valid, logits, jnp.float32(-1e30))
    m = jnp.max(masked, axis=1, keepdims=True)
    p = jnp.where(valid, jnp.exp(masked - m), 0.0)
    probs = p / jnp.sum(p, axis=1, keepdims=True)
    m1 = jnp.max(probs, axis=1, keepdims=True)
    i1 = jnp.min(jnp.where(probs == m1, cols, LANES), axis=1, keepdims=True)
    probs2 = jnp.where(cols == i1, -1.0, probs)
    m2 = jnp.max(probs2, axis=1, keepdims=True)
    i2 = jnp.min(jnp.where(probs2 == m2, cols, LANES), axis=1, keepdims=True)
    sel = (cols == i1) | (cols == i2)
    wv = jnp.where(sel & valid, probs, 0.0)
    sg = jnp.sum(jnp.where(cols == E, logits, 0.0), axis=1, keepdims=True)
    sgv = jax.nn.sigmoid(sg)
    w_ref[...] = jnp.where(cols == E, sgv, wv)


def _mlp_kernel(x_ref, g_ref, u_ref, d_ref, w_ref, o_ref):
    e = pl.program_id(0)
    k = pl.program_id(1)

    @pl.when((e == 0) & (k == 0))
    def _():
        o_ref[...] = jnp.zeros_like(o_ref)

    xb = x_ref[...]
    g = g_ref[0]
    u = u_ref[0]
    d = d_ref[0]
    h = lax.dot_general(xb, g, (((1,), (1,)), ((), ())),
                        preferred_element_type=jnp.float32)
    hu = lax.dot_general(xb, u, (((1,), (1,)), ((), ())),
                         preferred_element_type=jnp.float32)
    a = (h * jax.nn.sigmoid(h)) * hu
    contrib = lax.dot_general(a.astype(jnp.bfloat16), d,
                              (((1,), (1,)), ((), ())),
                              preferred_element_type=jnp.float32)
    cols = lax.broadcasted_iota(jnp.int32, (T, LANES), 1)
    we = jnp.sum(jnp.where(cols == e, w_ref[...], 0.0), axis=1, keepdims=True)
    o_ref[...] += we * contrib


@jax.jit
def kernel(hidden_states, router_w, gate_w, up_w, down_w,
           sh_gate_w, sh_up_w, sh_down_w, shared_gate_w):
    bsz, seq_len, hidden_size = hidden_states.shape
    x = hidden_states.reshape(T, D)

    rw_pad = jnp.zeros((LANES, D), jnp.float32)
    rw_pad = rw_pad.at[:E].set(router_w)
    rw_pad = rw_pad.at[E].set(shared_gate_w[0])

    w = pl.pallas_call(
        _router_kernel,
        out_shape=jax.ShapeDtypeStruct((T, LANES), jnp.float32),
    )(x, rw_pad)

    gw = jnp.concatenate([gate_w, sh_gate_w[None]], axis=0).astype(jnp.bfloat16)
    uw = jnp.concatenate([up_w, sh_up_w[None]], axis=0).astype(jnp.bfloat16)
    dw = jnp.concatenate([down_w, sh_down_w[None]], axis=0).astype(jnp.bfloat16)
    xb = x.astype(jnp.bfloat16)

    out = pl.pallas_call(
        _mlp_kernel,
        grid=(NE, K_CHUNKS),
        in_specs=[
            pl.BlockSpec((T, D), lambda e, k: (0, 0)),
            pl.BlockSpec((1, DFF_BLK, D), lambda e, k: (e, k, 0)),
            pl.BlockSpec((1, DFF_BLK, D), lambda e, k: (e, k, 0)),
            pl.BlockSpec((1, D, DFF_BLK), lambda e, k: (e, 0, k)),
            pl.BlockSpec((T, LANES), lambda e, k: (0, 0)),
        ],
        out_specs=pl.BlockSpec((T, D), lambda e, k: (0, 0)),
        out_shape=jax.ShapeDtypeStruct((T, D), jnp.float32),
        compiler_params=pltpu.CompilerParams(
            dimension_semantics=("arbitrary", "arbitrary")),
    )(xb, gw, uw, dw, w)

    return out.reshape(bsz, seq_len, hidden_size)
